# baseline (speedup 1.0000x reference)
import functools

import jax
import jax.numpy as jnp
from jax import lax
from jax.experimental import pallas as pl
from jax.experimental.pallas import tpu as pltpu
from jax.experimental.pallas import tpu_sc as plsc

_D = 128
_NC = 2
_NS = 16
_NW = _NC * _NS
_LANES = 16
_NBUF = 5
_SWAP_HALVES = False


def _linear_body(x_ref, w_ref, b_ref, z_ref):
    z_ref[...] = (
        lax.dot_general(
            x_ref[...], w_ref[...], (((1,), (0,)), ((), ())),
            precision=lax.Precision.HIGHEST,
            preferred_element_type=jnp.float32,
        )
        + b_ref[...]
    )


def _node_linear(x, w_t, b_row, row_block):
    n, d = x.shape
    grid = n // row_block
    return pl.pallas_call(
        _linear_body,
        grid=(grid,),
        in_specs=[
            pl.BlockSpec((row_block, d), lambda i: (i, 0)),
            pl.BlockSpec((d, d), lambda i: (0, 0)),
            pl.BlockSpec((1, d), lambda i: (0, 0)),
        ],
        out_specs=pl.BlockSpec((row_block, d), lambda i: (i, 0)),
        out_shape=jax.ShapeDtypeStruct((n, d), jnp.float32),
    )(x, w_t, b_row)


def _pack_table(z):
    n, d = z.shape
    zb = z.astype(jnp.bfloat16).reshape(n, d // 32, 2, 16)
    if _SWAP_HALVES:
        zb = zb[:, :, ::-1, :]
    zsw = zb.transpose(0, 1, 3, 2).reshape(n, d // 2, 2)
    return lax.bitcast_convert_type(zsw, jnp.int32)


def _make_edge_gather(n_edges, d, chunk):
    epw = n_edges // _NW
    n_chunks = epw // chunk
    n_groups = n_chunks // _NBUF
    assert n_chunks % _NBUF == 0 and n_groups >= 2
    mesh = plsc.VectorSubcoreMesh(core_axis_name="c", subcore_axis_name="s")

    scratch = (
        [pltpu.VMEM((epw,), jnp.int32)] * 2
        + [pltpu.VMEM((chunk, d // 2), jnp.int32)] * (2 * _NBUF)
        + [pltpu.VMEM((chunk, d), jnp.float32)] * _NBUF
        + [pltpu.SemaphoreType.DMA] * (3 * _NBUF)
    )

    @functools.partial(
        pl.kernel,
        mesh=mesh,
        out_type=jax.ShapeDtypeStruct((n_edges, d), jnp.float32),
        scratch_types=scratch,
        compiler_params=pltpu.CompilerParams(use_tc_tiling_on_sc=False),
    )
    def edge_gather(zp_hbm, idx0_hbm, idx1_hbm, out_hbm, *s):
        idx0_all, idx1_all = s[0], s[1]
        s = s[2:]
        rows0 = s[0:_NBUF]
        rows1 = s[_NBUF:2 * _NBUF]
        outb = s[2 * _NBUF:3 * _NBUF]
        g0sem = s[3 * _NBUF:4 * _NBUF]
        g1sem = s[4 * _NBUF:5 * _NBUF]
        osem = s[5 * _NBUF:6 * _NBUF]

        wid = lax.axis_index("s") * _NC + lax.axis_index("c")
        base0 = pl.multiple_of(wid * epw, 8)
        pltpu.sync_copy(idx0_hbm.at[pl.ds(base0, epw)], idx0_all)
        pltpu.sync_copy(idx1_hbm.at[pl.ds(base0, epw)], idx1_all)

        def gather_cps(b, k):
            off = pl.multiple_of(k * chunk, 8)
            cp0 = pltpu.make_async_copy(
                zp_hbm.at[idx0_all.at[pl.ds(off, chunk)]], rows0[b], g0sem[b])
            cp1 = pltpu.make_async_copy(
                zp_hbm.at[idx1_all.at[pl.ds(off, chunk)]], rows1[b], g1sem[b])
            return cp0, cp1

        def store_cp(b, k):
            off = pl.multiple_of(base0 + k * chunk, 8)
            return pltpu.make_async_copy(
                outb[b], out_hbm.at[pl.ds(off, chunk)], osem[b])

        def fire_gathers(b, k):
            cp0, cp1 = gather_cps(b, k)
            cp0.start()
            cp1.start()

        def do_chunk(k, b, refire, wait_prev_store):
            cp0, cp1 = gather_cps(b, k)
            cp0.wait()
            cp1.wait()
            if wait_prev_store:
                store_cp(b, k - _NBUF).wait()

            hmask = jnp.int32(-65536)

            def expand(w, high):
                bits = (w & hmask) if high else (w << 16)
                return lax.bitcast_convert_type(bits, jnp.float32)

            def add_body(i, c):
                for j in range(d // 32):
                    sl = pl.ds(j * _LANES, _LANES)
                    w0 = rows0[b][i, sl]
                    w1 = rows1[b][i, sl]
                    outb[b][i, pl.ds(j * 32, 16)] = (
                        expand(w0, False) + expand(w1, False))
                    outb[b][i, pl.ds(j * 32 + 16, 16)] = (
                        expand(w0, True) + expand(w1, True))
                return c

            lax.fori_loop(0, chunk, add_body, 0, unroll=2)
            store_cp(b, k).start()
            if refire:
                fire_gathers(b, k + _NBUF)

        for b in range(_NBUF):
            fire_gathers(b, b)
        for b in range(_NBUF):
            do_chunk(b, b, refire=True, wait_prev_store=False)
        def group_body(g, carry):
            k0 = g * _NBUF
            for b in range(_NBUF):
                do_chunk(k0 + b, b, refire=True, wait_prev_store=True)
            return carry

        lax.fori_loop(1, n_groups - 1, group_body, 0)
        k0 = (n_groups - 1) * _NBUF
        for b in range(_NBUF):
            do_chunk(k0 + b, b, refire=False, wait_prev_store=True)
        for b in range(_NBUF):
            store_cp(b, k0 + b).wait()

    return edge_gather


def kernel(x, edges, W, b):
    n, d = x.shape
    n_edges = edges.shape[0]
    e32 = edges.astype(jnp.int32)
    idx0 = e32[:, 0]
    idx1 = e32[:, 1]
    w_t = (0.5 * W).T.astype(jnp.float32)
    b_row = (0.5 * b).reshape(1, d).astype(jnp.float32)
    z = _node_linear(x, w_t, b_row, row_block=1000)
    zp = _pack_table(z)
    edge_gather = _make_edge_gather(n_edges, d, chunk=80)
    return edge_gather(zp, idx0, idx1)


# speedup vs baseline: 1.8044x; 1.8044x over previous
import functools

import jax
import jax.numpy as jnp
from jax import lax
from jax.experimental import pallas as pl
from jax.experimental.pallas import tpu as pltpu
from jax.experimental.pallas import tpu_sc as plsc

_D = 128
_NC = 2
_NS = 16
_NW = _NC * _NS
_LANES = 16
_NBUF = 5
_SWAP_HALVES = False


def _linear_body(x_ref, w_ref, b_ref, z_ref):
    z_ref[...] = (
        lax.dot_general(
            x_ref[...], w_ref[...], (((1,), (0,)), ((), ())),
            precision=lax.Precision.HIGHEST,
            preferred_element_type=jnp.float32,
        )
        + b_ref[...]
    )


def _node_linear(x, w_t, b_row, row_block):
    n, d = x.shape
    grid = n // row_block
    return pl.pallas_call(
        _linear_body,
        grid=(grid,),
        in_specs=[
            pl.BlockSpec((row_block, d), lambda i: (i, 0)),
            pl.BlockSpec((d, d), lambda i: (0, 0)),
            pl.BlockSpec((1, d), lambda i: (0, 0)),
        ],
        out_specs=pl.BlockSpec((row_block, d), lambda i: (i, 0)),
        out_shape=jax.ShapeDtypeStruct((n, d), jnp.float32),
    )(x, w_t, b_row)


def _half_perm(d):
    import numpy as np
    blocks = np.arange(d).reshape(d // 32, 2, 16)
    return np.concatenate([blocks[:, 0, :].ravel(), blocks[:, 1, :].ravel()])


def _pack_table(z):
    n, d = z.shape
    zb = lax.bitcast_convert_type(
        z.astype(jnp.bfloat16), jnp.uint16).astype(jnp.uint32)
    lo, hi = zb[:, :d // 2], zb[:, d // 2:]
    if _SWAP_HALVES:
        lo, hi = hi, lo
    return lax.bitcast_convert_type(lo | (hi << 16), jnp.int32)


def _make_edge_gather(n_edges, d, chunk):
    epw = n_edges // _NW
    n_chunks = epw // chunk
    n_groups = n_chunks // _NBUF
    assert n_chunks % _NBUF == 0 and n_groups >= 2
    mesh = plsc.VectorSubcoreMesh(core_axis_name="c", subcore_axis_name="s")

    scratch = (
        [pltpu.VMEM((epw,), jnp.int32)] * 2
        + [pltpu.VMEM((chunk, d // 2), jnp.int32)] * (2 * _NBUF)
        + [pltpu.VMEM((chunk, d), jnp.float32)] * _NBUF
        + [pltpu.SemaphoreType.DMA] * (3 * _NBUF)
    )

    @functools.partial(
        pl.kernel,
        mesh=mesh,
        out_type=jax.ShapeDtypeStruct((n_edges, d), jnp.float32),
        scratch_types=scratch,
        compiler_params=pltpu.CompilerParams(use_tc_tiling_on_sc=False),
    )
    def edge_gather(zp_hbm, idx0_hbm, idx1_hbm, out_hbm, *s):
        idx0_all, idx1_all = s[0], s[1]
        s = s[2:]
        rows0 = s[0:_NBUF]
        rows1 = s[_NBUF:2 * _NBUF]
        outb = s[2 * _NBUF:3 * _NBUF]
        g0sem = s[3 * _NBUF:4 * _NBUF]
        g1sem = s[4 * _NBUF:5 * _NBUF]
        osem = s[5 * _NBUF:6 * _NBUF]

        wid = lax.axis_index("s") * _NC + lax.axis_index("c")
        base0 = pl.multiple_of(wid * epw, 8)
        pltpu.sync_copy(idx0_hbm.at[pl.ds(base0, epw)], idx0_all)
        pltpu.sync_copy(idx1_hbm.at[pl.ds(base0, epw)], idx1_all)

        def gather_cps(b, k):
            off = pl.multiple_of(k * chunk, 8)
            cp0 = pltpu.make_async_copy(
                zp_hbm.at[idx0_all.at[pl.ds(off, chunk)]], rows0[b], g0sem[b])
            cp1 = pltpu.make_async_copy(
                zp_hbm.at[idx1_all.at[pl.ds(off, chunk)]], rows1[b], g1sem[b])
            return cp0, cp1

        def store_cp(b, k):
            off = pl.multiple_of(base0 + k * chunk, 8)
            return pltpu.make_async_copy(
                outb[b], out_hbm.at[pl.ds(off, chunk)], osem[b])

        def fire_gathers(b, k):
            cp0, cp1 = gather_cps(b, k)
            cp0.start()
            cp1.start()

        def do_chunk(k, b, refire, wait_prev_store):
            cp0, cp1 = gather_cps(b, k)
            cp0.wait()
            cp1.wait()
            if wait_prev_store:
                store_cp(b, k - _NBUF).wait()

            hmask = jnp.int32(-65536)

            def expand(w, high):
                bits = (w & hmask) if high else (w << 16)
                return lax.bitcast_convert_type(bits, jnp.float32)

            @functools.partial(plsc.parallel_loop, 0, chunk, unroll=4)
            def add_body(i):
                for j in range(d // 32):
                    sl = pl.ds(j * _LANES, _LANES)
                    w0 = rows0[b][i, sl]
                    w1 = rows1[b][i, sl]
                    outb[b][i, pl.ds(j * 32, 16)] = (
                        expand(w0, False) + expand(w1, False))
                    outb[b][i, pl.ds(j * 32 + 16, 16)] = (
                        expand(w0, True) + expand(w1, True))
            store_cp(b, k).start()
            if refire:
                fire_gathers(b, k + _NBUF)

        for b in range(_NBUF):
            fire_gathers(b, b)
        for b in range(_NBUF):
            do_chunk(b, b, refire=True, wait_prev_store=False)
        def group_body(g, carry):
            k0 = g * _NBUF
            for b in range(_NBUF):
                do_chunk(k0 + b, b, refire=True, wait_prev_store=True)
            return carry

        lax.fori_loop(1, n_groups - 1, group_body, 0)
        k0 = (n_groups - 1) * _NBUF
        for b in range(_NBUF):
            do_chunk(k0 + b, b, refire=False, wait_prev_store=True)
        for b in range(_NBUF):
            store_cp(b, k0 + b).wait()

    return edge_gather


def kernel(x, edges, W, b):
    n, d = x.shape
    n_edges = edges.shape[0]
    e32 = edges.astype(jnp.int32)
    idx0 = e32[:, 0]
    idx1 = e32[:, 1]
    w_t = (0.5 * W).T.astype(jnp.float32)
    b_row = (0.5 * b).reshape(1, d).astype(jnp.float32)
    perm = _half_perm(d)
    w_t = w_t[:, perm]
    b_row = b_row[:, perm]
    z = _node_linear(x, w_t, b_row, row_block=1000)
    zp = _pack_table(z)
    edge_gather = _make_edge_gather(n_edges, d, chunk=80)
    return edge_gather(zp, idx0, idx1)
